# Initial kernel scaffold; baseline (speedup 1.0000x reference)
import jax
import jax.numpy as jnp
from jax.experimental import pallas as pl


def kernel(pos_x, neg_x, ent_table, rel_table, W1, b1, W2, b2):
    raise NotImplementedError("write your pallas kernel here")


# speedup vs baseline: 1.1050x; 1.1050x over previous
import jax
import jax.numpy as jnp
from jax import lax
from jax.experimental import pallas as pl
from jax.experimental.pallas import tpu as pltpu
from jax.experimental.pallas import tpu_sc as plsc

BATCH = 16384
DEPTH = 64
HIDDEN = 128
MARGIN = 1.0

CHUNK = 128
PAIR = 2 * DEPTH
N_ENT_ROWS = 4 * BATCH
N_REL_ROWS = 2 * BATCH

BB = 2048


def _sc_gather_body(ent_hbm, rel_hbm, eidx_hbm, ridx_hbm, ent_out, rel_out,
                    idx_v, rows_v, sem):
    nc = plsc.get_sparse_core_info().num_cores
    wid = lax.axis_index("s") * nc + lax.axis_index("c")
    nw = nc * plsc.get_sparse_core_info().num_subcores

    def chunk_loop(table, idx2d, out, chunks_per_w):
        base = wid * chunks_per_w

        def body(j, carry):
            row = base + j
            pltpu.sync_copy(idx2d.at[row], idx_v)
            pltpu.async_copy(table.at[idx_v], rows_v, sem).wait()
            pltpu.sync_copy(rows_v, out.at[pl.ds(row * CHUNK, CHUNK)])
            return carry

        lax.fori_loop(0, chunks_per_w, body, 0)

    chunk_loop(ent_hbm, eidx_hbm, ent_out, N_ENT_ROWS // (nw * CHUNK))
    chunk_loop(rel_hbm, ridx_hbm, rel_out, N_REL_ROWS // (nw * CHUNK))


def _tc_body(hp, tp, rp, hg, tg, rg, php, ptp, prp, phg, ptg, prg,
             w1, b1, w2, b2, out_ref):
    i = pl.program_id(0)

    w1v = w1[...]
    w1a = w1v[:DEPTH]
    w1b = w1v[DEPTH:]
    w2v = w2[...]
    b1v = b1[...]
    b2v = b2[...]

    def pick(pairs_ref, par_ref):
        g = pairs_ref[...]
        p = par_ref[...] > 0
        return jnp.where(p, g[:, DEPTH:], g[:, :DEPTH])

    def normalize(x):
        ss = jnp.sum(x * x, axis=1, keepdims=True)
        return x / jnp.maximum(jnp.sqrt(ss), 1e-12)

    def score(h, t, r):
        hn = normalize(h)
        tn = normalize(t)
        hid = lax.dot_general(hn, w1a, (((1,), (0,)), ((), ())),
                              preferred_element_type=jnp.float32)
        hid += lax.dot_general(r, w1b, (((1,), (0,)), ((), ())),
                               preferred_element_type=jnp.float32)
        hid = jnp.maximum(hid + b1v, 0.0)
        out = lax.dot_general(hid, w2v, (((1,), (0,)), ((), ())),
                              preferred_element_type=jnp.float32) + b2v
        d = out - tn
        return jnp.sqrt(jnp.sum(d * d, axis=1))

    ps = score(pick(hp, php), pick(tp, ptp), pick(rp, prp))
    ns = score(pick(hg, phg), pick(tg, ptg), pick(rg, prg))
    part = jnp.sum(jnp.maximum(MARGIN + ps - ns, 0.0)).reshape(1, 1)

    @pl.when(i == 0)
    def _():
        out_ref[...] = jnp.zeros((1, 1), jnp.float32)

    out_ref[...] += part

    @pl.when(i == pl.num_programs(0) - 1)
    def _():
        out_ref[...] = out_ref[...] * (1.0 / BATCH)


def _tc_score(ent_pairs, rel_pairs, p_ent, p_rel, W1, b1, W2, b2):
    nb = BATCH // BB
    grid = (nb,)
    pair_spec = lambda off: pl.BlockSpec((BB, PAIR), lambda i, o=off: (i + o, 0))
    par_spec = lambda off: pl.BlockSpec((BB, 1), lambda i, o=off: (i + o, 0))
    res = pl.pallas_call(
        _tc_body,
        grid=grid,
        in_specs=[
            pair_spec(0),
            pair_spec(nb),
            pair_spec(0),
            pair_spec(2 * nb),
            pair_spec(3 * nb),
            pair_spec(nb),
            par_spec(0), par_spec(nb), par_spec(0),
            par_spec(2 * nb), par_spec(3 * nb), par_spec(nb),
            pl.BlockSpec((2 * DEPTH, HIDDEN), lambda i: (0, 0)),
            pl.BlockSpec((1, HIDDEN), lambda i: (0, 0)),
            pl.BlockSpec((HIDDEN, DEPTH), lambda i: (0, 0)),
            pl.BlockSpec((1, DEPTH), lambda i: (0, 0)),
        ],
        out_specs=pl.BlockSpec((1, 1), lambda i: (0, 0)),
        out_shape=jax.ShapeDtypeStruct((1, 1), jnp.float32),
    )(ent_pairs, ent_pairs, rel_pairs, ent_pairs, ent_pairs, rel_pairs,
      p_ent, p_ent, p_rel, p_ent, p_ent, p_rel,
      W1, b1.reshape(1, HIDDEN), W2, b2.reshape(1, DEPTH))
    return res[0, 0]


def kernel(pos_x, neg_x, ent_table, rel_table, W1, b1, W2, b2):
    n_ent = ent_table.shape[0]
    n_rel = rel_table.shape[0]
    ent2 = ent_table.reshape(n_ent // 2, PAIR)
    rel2 = rel_table.reshape(n_rel // 2, PAIR)

    eidx = jnp.concatenate(
        [pos_x[:, 0], pos_x[:, 1], neg_x[:, 0], neg_x[:, 1]])
    ridx = jnp.concatenate([pos_x[:, 2], neg_x[:, 2]])
    p_ent = (eidx & 1).astype(jnp.int32).reshape(N_ENT_ROWS, 1)
    p_rel = (ridx & 1).astype(jnp.int32).reshape(N_REL_ROWS, 1)
    eidx2 = (eidx >> 1).reshape(N_ENT_ROWS // CHUNK, CHUNK)
    ridx2 = (ridx >> 1).reshape(N_REL_ROWS // CHUNK, CHUNK)

    sc_gather = pl.kernel(
        _sc_gather_body,
        out_type=[
            jax.ShapeDtypeStruct((N_ENT_ROWS, PAIR), jnp.float32),
            jax.ShapeDtypeStruct((N_REL_ROWS, PAIR), jnp.float32),
        ],
        mesh=plsc.VectorSubcoreMesh(core_axis_name="c", subcore_axis_name="s"),
        scratch_types=[
            pltpu.VMEM((CHUNK,), jnp.int32),
            pltpu.VMEM((CHUNK, PAIR), jnp.float32),
            pltpu.SemaphoreType.DMA,
        ],
    )
    ent_pairs, rel_pairs = sc_gather(ent2, rel2, eidx2, ridx2)
    return _tc_score(ent_pairs, rel_pairs, p_ent, p_rel, W1, b1, W2, b2)


# speedup vs baseline: 1.1487x; 1.0396x over previous
import jax
import jax.numpy as jnp
from jax import lax
from jax.experimental import pallas as pl
from jax.experimental.pallas import tpu as pltpu
from jax.experimental.pallas import tpu_sc as plsc

BATCH = 16384
DEPTH = 64
HIDDEN = 128
MARGIN = 1.0

CHUNK = 128
N_ENT_ROWS = 4 * BATCH
N_REL_ROWS = 2 * BATCH

BB = 2048


def _sc_gather_body(ent_hbm, rel_hbm, eidx_hbm, ridx_hbm, ent_out, rel_out,
                    idx_v, rows_v, sem):
    nc = plsc.get_sparse_core_info().num_cores
    wid = lax.axis_index("s") * nc + lax.axis_index("c")
    nw = nc * plsc.get_sparse_core_info().num_subcores

    def chunk_loop(table, idx2d, out, chunks_per_w):
        base = wid * chunks_per_w

        def body(j, carry):
            row = base + j
            pltpu.sync_copy(idx2d.at[row], idx_v)
            pltpu.async_copy(table.at[idx_v], rows_v, sem).wait()
            pltpu.sync_copy(rows_v, out.at[pl.ds(row * CHUNK, CHUNK)])
            return carry

        lax.fori_loop(0, chunks_per_w, body, 0)

    chunk_loop(ent_hbm, eidx_hbm, ent_out, N_ENT_ROWS // (nw * CHUNK))
    chunk_loop(rel_hbm, ridx_hbm, rel_out, N_REL_ROWS // (nw * CHUNK))


def _tc_body(hp, tp, rp, hg, tg, rg, w1, b1, w2, b2, out_ref):
    i = pl.program_id(0)

    w1v = w1[...]
    w1a = w1v[:DEPTH]
    w1b = w1v[DEPTH:]
    w2v = w2[...]
    b1v = b1[...]
    b2v = b2[...]

    def normalize(x):
        ss = jnp.sum(x * x, axis=1, keepdims=True)
        return x / jnp.maximum(jnp.sqrt(ss), 1e-12)

    def score(h, t, r):
        hn = normalize(h)
        tn = normalize(t)
        hid = lax.dot_general(hn, w1a, (((1,), (0,)), ((), ())),
                              preferred_element_type=jnp.float32)
        hid += lax.dot_general(r, w1b, (((1,), (0,)), ((), ())),
                               preferred_element_type=jnp.float32)
        hid = jnp.maximum(hid + b1v, 0.0)
        out = lax.dot_general(hid, w2v, (((1,), (0,)), ((), ())),
                              preferred_element_type=jnp.float32) + b2v
        d = out - tn
        return jnp.sqrt(jnp.sum(d * d, axis=1))

    ps = score(hp[...], tp[...], rp[...])
    ns = score(hg[...], tg[...], rg[...])
    part = jnp.sum(jnp.maximum(MARGIN + ps - ns, 0.0)).reshape(1, 1)

    @pl.when(i == 0)
    def _():
        out_ref[...] = jnp.zeros((1, 1), jnp.float32)

    out_ref[...] += part

    @pl.when(i == pl.num_programs(0) - 1)
    def _():
        out_ref[...] = out_ref[...] * (1.0 / BATCH)


def _tc_score(ent_rows, rel_rows, W1, b1, W2, b2):
    nb = BATCH // BB
    grid = (nb,)
    row_spec = lambda off: pl.BlockSpec((BB, DEPTH), lambda i, o=off: (i + o, 0))
    res = pl.pallas_call(
        _tc_body,
        grid=grid,
        in_specs=[
            row_spec(0),
            row_spec(nb),
            pl.BlockSpec((BB, DEPTH), lambda i: (i, 0)),
            row_spec(2 * nb),
            row_spec(3 * nb),
            pl.BlockSpec((BB, DEPTH), lambda i: (i + nb, 0)),
            pl.BlockSpec((2 * DEPTH, HIDDEN), lambda i: (0, 0)),
            pl.BlockSpec((1, HIDDEN), lambda i: (0, 0)),
            pl.BlockSpec((HIDDEN, DEPTH), lambda i: (0, 0)),
            pl.BlockSpec((1, DEPTH), lambda i: (0, 0)),
        ],
        out_specs=pl.BlockSpec((1, 1), lambda i: (0, 0)),
        out_shape=jax.ShapeDtypeStruct((1, 1), jnp.float32),
    )(ent_rows, ent_rows, rel_rows, ent_rows, ent_rows, rel_rows,
      W1, b1.reshape(1, HIDDEN), W2, b2.reshape(1, DEPTH))
    return res[0, 0]


def kernel(pos_x, neg_x, ent_table, rel_table, W1, b1, W2, b2):
    eidx = jnp.concatenate(
        [pos_x[:, 0], pos_x[:, 1], neg_x[:, 0], neg_x[:, 1]]
    ).reshape(N_ENT_ROWS // CHUNK, CHUNK)
    ridx = jnp.concatenate(
        [pos_x[:, 2], neg_x[:, 2]]
    ).reshape(N_REL_ROWS // CHUNK, CHUNK)

    sc_gather = pl.kernel(
        _sc_gather_body,
        out_type=[
            jax.ShapeDtypeStruct((N_ENT_ROWS, DEPTH), jnp.float32),
            jax.ShapeDtypeStruct((N_REL_ROWS, DEPTH), jnp.float32),
        ],
        mesh=plsc.VectorSubcoreMesh(core_axis_name="c", subcore_axis_name="s"),
        compiler_params=pltpu.CompilerParams(use_tc_tiling_on_sc=False),
        scratch_types=[
            pltpu.VMEM((CHUNK,), jnp.int32),
            pltpu.VMEM((CHUNK, DEPTH), jnp.float32),
            pltpu.SemaphoreType.DMA,
        ],
    )
    ent_rows, rel_rows = sc_gather(ent_table, rel_table, eidx, ridx)
    return _tc_score(ent_rows, rel_rows, W1, b1, W2, b2)
